# baseline (speedup 1.0000x reference)
import jax
import jax.numpy as jnp
from jax.experimental import pallas as pl
from jax.experimental.pallas import tpu as pltpu

_B, _H, _D, _S = 8, 8, 128, 2048


def _body(idx_ref, key_ref, val_ref, ck_ref, cv_ref, ok_ref, ov_ref):
    idx = idx_ref[0]
    b = pl.program_id(0)
    h = pl.program_id(1)
    r = b * _H + h
    ok_ref[...] = ck_ref[...].reshape(1, _S, _D)
    ov_ref[...] = cv_ref[...].reshape(1, _S, _D)
    ok_ref[:, pl.ds(idx, 1), :] = key_ref[pl.ds(r, 1), :].reshape(1, 1, _D)
    ov_ref[:, pl.ds(idx, 1), :] = val_ref[pl.ds(r, 1), :].reshape(1, 1, _D)


def kernel(key, value, cached_key, cached_value, cache_index):
    idx = jnp.asarray(cache_index, jnp.int32).reshape(1)
    ck2 = cached_key.reshape(_S, _H * _B * _D)
    cv2 = cached_value.reshape(_S, _H * _B * _D)
    k2 = key.reshape(_B * _H, _D)
    v2 = value.reshape(_B * _H, _D)
    out_shape = [jax.ShapeDtypeStruct((_B, _S, _H * _D), jnp.float32)] * 2
    ok, ov = pl.pallas_call(
        _body,
        grid=(_B, _H),
        in_specs=[
            pl.BlockSpec(memory_space=pltpu.SMEM),
            pl.BlockSpec((_B * _H, _D), lambda b, h: (0, 0)),
            pl.BlockSpec((_B * _H, _D), lambda b, h: (0, 0)),
            pl.BlockSpec((_S, _D), lambda b, h: (0, h * _B + b)),
            pl.BlockSpec((_S, _D), lambda b, h: (0, h * _B + b)),
        ],
        out_specs=[
            pl.BlockSpec((1, _S, _D), lambda b, h: (b, 0, h)),
            pl.BlockSpec((1, _S, _D), lambda b, h: (b, 0, h)),
        ],
        out_shape=out_shape,
    )(idx, k2, v2, ck2, cv2)
    return ok.reshape(_B, _S, _H, _D), ov.reshape(_B, _S, _H, _D)


# speedup vs baseline: 4.0616x; 4.0616x over previous
import jax
import jax.numpy as jnp
from jax.experimental import pallas as pl
from jax.experimental.pallas import tpu as pltpu

_B, _H, _D, _S = 8, 8, 128, 2048
_X = _S * _H
_XBLK = 512


def _body(idx_ref, key_ref, val_ref, ck_ref, cv_ref, ok_ref, ov_ref):
    idx = idx_ref[0]
    j = pl.program_id(0)
    ok_ref[...] = jnp.transpose(ck_ref[...], (1, 0, 2))
    ov_ref[...] = jnp.transpose(cv_ref[...], (1, 0, 2))
    xtok = idx * _H

    @pl.when(j == xtok // _XBLK)
    def _():
        loc = xtok % _XBLK
        ok_ref[:, pl.ds(loc, _H), :] = key_ref[...]
        ov_ref[:, pl.ds(loc, _H), :] = val_ref[...]


def kernel(key, value, cached_key, cached_value, cache_index):
    idx = jnp.asarray(cache_index, jnp.int32).reshape(1)
    ck3 = cached_key.reshape(_X, _B, _D)
    cv3 = cached_value.reshape(_X, _B, _D)
    k3 = key.reshape(_B, _H, _D)
    v3 = value.reshape(_B, _H, _D)
    out_shape = [jax.ShapeDtypeStruct((_B, _X, _D), jnp.float32)] * 2
    ok, ov = pl.pallas_call(
        _body,
        grid=(_X // _XBLK,),
        in_specs=[
            pl.BlockSpec(memory_space=pltpu.SMEM),
            pl.BlockSpec((_B, _H, _D), lambda j: (0, 0, 0)),
            pl.BlockSpec((_B, _H, _D), lambda j: (0, 0, 0)),
            pl.BlockSpec((_XBLK, _B, _D), lambda j: (j, 0, 0)),
            pl.BlockSpec((_XBLK, _B, _D), lambda j: (j, 0, 0)),
        ],
        out_specs=[
            pl.BlockSpec((_B, _XBLK, _D), lambda j: (0, j, 0)),
            pl.BlockSpec((_B, _XBLK, _D), lambda j: (0, j, 0)),
        ],
        out_shape=out_shape,
    )(idx, k3, v3, ck3, cv3)
    return ok.reshape(_B, _S, _H, _D), ov.reshape(_B, _S, _H, _D)


# speedup vs baseline: 4.1204x; 1.0145x over previous
import jax
import jax.numpy as jnp
from jax.experimental import pallas as pl
from jax.experimental.pallas import tpu as pltpu

_B, _H, _D, _S = 8, 8, 128, 2048
_X = _S * _H
_XBLK = 1024


def _body(idx_ref, key_ref, val_ref, ck_ref, cv_ref, ok_ref, ov_ref):
    idx = idx_ref[0]
    j = pl.program_id(0)
    ok_ref[...] = jnp.transpose(ck_ref[...], (1, 0, 2))
    ov_ref[...] = jnp.transpose(cv_ref[...], (1, 0, 2))
    xtok = idx * _H

    @pl.when(j == xtok // _XBLK)
    def _():
        loc = xtok % _XBLK
        ok_ref[:, pl.ds(loc, _H), :] = key_ref[...]
        ov_ref[:, pl.ds(loc, _H), :] = val_ref[...]


def kernel(key, value, cached_key, cached_value, cache_index):
    idx = jnp.asarray(cache_index, jnp.int32).reshape(1)
    ck3 = cached_key.reshape(_X, _B, _D)
    cv3 = cached_value.reshape(_X, _B, _D)
    k3 = key.reshape(_B, _H, _D)
    v3 = value.reshape(_B, _H, _D)
    out_shape = [jax.ShapeDtypeStruct((_B, _X, _D), jnp.float32)] * 2
    ok, ov = pl.pallas_call(
        _body,
        grid=(_X // _XBLK,),
        in_specs=[
            pl.BlockSpec(memory_space=pltpu.SMEM),
            pl.BlockSpec((_B, _H, _D), lambda j: (0, 0, 0)),
            pl.BlockSpec((_B, _H, _D), lambda j: (0, 0, 0)),
            pl.BlockSpec((_XBLK, _B, _D), lambda j: (j, 0, 0)),
            pl.BlockSpec((_XBLK, _B, _D), lambda j: (j, 0, 0)),
        ],
        out_specs=[
            pl.BlockSpec((_B, _XBLK, _D), lambda j: (0, j, 0)),
            pl.BlockSpec((_B, _XBLK, _D), lambda j: (0, j, 0)),
        ],
        out_shape=out_shape,
    )(idx, k3, v3, ck3, cv3)
    return ok.reshape(_B, _S, _H, _D), ov.reshape(_B, _S, _H, _D)
